# Initial kernel scaffold; baseline (speedup 1.0000x reference)
import jax
import jax.numpy as jnp
from jax.experimental import pallas as pl


def kernel(features, W_conv, b_conv):
    raise NotImplementedError("write your pallas kernel here")


# speedup vs baseline: 1.1969x; 1.1969x over previous
import jax
import jax.numpy as jnp
from jax.experimental import pallas as pl
from jax.experimental.pallas import tpu as pltpu

_TOP_K = 0.1
_UNIFORM_R = 0.5
_BH = 32


def _encode_body(w_ref, b_ref, x_ref, enc_ref, comp_ref):
    x = x_ref[0]
    c = x.shape[0]
    xm = x.reshape(c, -1)
    y = jnp.dot(w_ref[...], xm, preferred_element_type=jnp.float32)
    z = y + b_ref[...]
    enc_ref[0] = z.reshape(x.shape)
    t = z.reshape(8, c // 8, -1)
    s = t[0]
    for i in range(1, 8):
        s = s + t[i]
    u = s[:4] + s[4:]
    v = u[:2] + u[2:]
    w = v[0:1] + v[1:2]
    comp_ref[0] = w.reshape(x.shape[1:])


def _encode(features, W_conv, b_conv):
    N, C, H, W = features.shape
    grid = (N, H // _BH)
    return pl.pallas_call(
        _encode_body,
        grid=grid,
        in_specs=[
            pl.BlockSpec((C, C), lambda n, h: (0, 0)),
            pl.BlockSpec((C, 1), lambda n, h: (0, 0)),
            pl.BlockSpec((1, C, _BH, W), lambda n, h: (n, 0, h, 0)),
        ],
        out_specs=[
            pl.BlockSpec((1, C, _BH, W), lambda n, h: (n, 0, h, 0)),
            pl.BlockSpec((1, _BH, W), lambda n, h: (n, h, 0)),
        ],
        out_shape=[
            jax.ShapeDtypeStruct((N, C, H, W), jnp.float32),
            jax.ShapeDtypeStruct((N, H, W), jnp.float32),
        ],
        compiler_params=pltpu.CompilerParams(
            dimension_semantics=("parallel", "arbitrary"),
        ),
    )(W_conv, b_conv.reshape(C, 1), features)


def kernel(features, W_conv, b_conv):
    N, C, H, W = features.shape
    encoded, compressed = _encode(features, W_conv, b_conv)
    k_sel = int(H * W * _TOP_K)
    flat = compressed.reshape(N, H * W)
    _, indices = jax.lax.top_k(flat, k_sel)
    k_new = int(k_sel * _UNIFORM_R)
    perm = jax.random.permutation(jax.random.key(42), N)
    indices = indices[perm][:, :k_new]
    h = indices // W
    w = indices % W
    sparse_indices = jnp.stack([h, w], axis=-1).astype(jnp.int32)
    sparse_features = jnp.take_along_axis(flat, indices, axis=1)
    return sparse_features, sparse_indices, encoded
